# baseline (speedup 1.0000x reference)
import functools

import jax
import jax.numpy as jnp
from jax import lax
from jax.experimental import pallas as pl
from jax.experimental.pallas import tpu as pltpu
from jax.experimental.pallas import tpu_sc as plsc

N = 10000
E = 160000
DIN = 256
H = 512
NCLS = 10

NC = 2
NS = 16
NW = NC * NS
EPW = E // NW
ECH = 2048
NCH_FULL = E // ECH
ECH_TAIL = E - NCH_FULL * ECH
FCAP = ECH + 64

R0 = 313
R0_LAST = N - 31 * R0
R1 = 157
R1_LAST = N - 63 * R1
K0 = 64
K1 = 32
NCH = E // ECH

_mesh = plsc.VectorSubcoreMesh(core_axis_name="c", subcore_axis_name="s")
_scp = pltpu.CompilerParams(needs_layout_passes=False)


def _iota16():
    return lax.broadcasted_iota(jnp.int32, (16,), 0)


@functools.partial(
    pl.kernel,
    out_type=jax.ShapeDtypeStruct((NC * N,), jnp.float32),
    mesh=_mesh,
    compiler_params=_scp,
    scratch_types=[
        pltpu.VMEM((EPW + 24,), jnp.int32),
        pltpu.VMEM((EPW + 24,), jnp.float32),
        pltpu.VMEM_SHARED((N,), jnp.float32),
    ],
)
def _deg_sc(dst_hbm, degp_hbm, idx_v, val_v, acc_sh):
    c = lax.axis_index("c")
    s = lax.axis_index("s")
    wid = c * NS + s
    npad = EPW + 24
    iota = _iota16()

    def zfill(i, _):
        val_v[pl.ds(i * 16, 16)] = jnp.zeros((16,), jnp.float32)
        return 0
    lax.fori_loop(0, npad // 16, zfill, 0)

    @pl.when(s < 2)
    def _():
        pltpu.sync_copy(val_v.at[pl.ds(0, EPW)], acc_sh.at[pl.ds(s * EPW, EPW)])

    plsc.subcore_barrier()

    pltpu.sync_copy(dst_hbm.at[pl.ds(wid * EPW, EPW)], idx_v.at[pl.ds(0, EPW)])

    def vfill(i, _):
        pos = iota + i * 16
        val_v[pl.ds(i * 16, 16)] = jnp.where(pos < EPW, 1.0, 0.0).astype(jnp.float32)
        return 0
    lax.fori_loop(0, npad // 16, vfill, 0)
    for off in range(EPW // 16 * 16, npad, 16):
        pos = iota + off
        old = idx_v[pl.ds(off, 16)]
        idx_v[pl.ds(off, 16)] = jnp.where(pos < EPW, old, 0)

    pltpu.sync_copy(val_v, acc_sh.at[idx_v], add=True)
    plsc.subcore_barrier()

    @pl.when(s < 2)
    def _():
        pltpu.sync_copy(acc_sh.at[pl.ds(s * EPW, EPW)], val_v.at[pl.ds(0, EPW)])
        pltpu.sync_copy(val_v.at[pl.ds(0, EPW)],
                        degp_hbm.at[pl.ds(c * N + s * EPW, EPW)])


def _filter_chunk(schunk, dchunk, base, fsrc, fdst, nvec, lo, hi, kpad,
                  trash):
    iota = _iota16()

    @plsc.parallel_loop(0, nvec, unroll=8, carry=jnp.int32(0))
    def cnt(i, cnt):
        d = dchunk[pl.ds(base + i * 16, 16)]
        sv = schunk[pl.ds(base + i * 16, 16)]
        m = (d >= lo) & (d < hi)
        mi = m.astype(jnp.int32)
        inc = plsc.cumsum(mi)
        pos = cnt + inc - mi
        plsc.store_scatter(fdst, [pos], d - lo, mask=m)
        plsc.store_scatter(fsrc, [pos], sv, mask=m)
        return cnt + plsc.all_reduce_population_count(m)[0]
    for t in range(kpad // 16):
        fsrc[pl.ds(cnt + t * 16, 16)] = iota
        fdst[pl.ds(cnt + t * 16, 16)] = jnp.full((16,), trash, jnp.int32)
    return cnt


def _agg_body(src_hbm, dst_hbm, m2d, m1d, out1d, sch2, dch2, fsrc, fdst,
              stage2, acc1, sem, semg, lo, rows, d, kblk):
    words = rows * d
    pltpu.sync_copy(m1d.at[pl.ds(lo * d, words)], acc1.at[pl.ds(0, words)])
    hi = lo + rows

    def gather_issue(b, bsel):
        pltpu.async_copy(m2d.at[fsrc.at[pl.ds(b * kblk, kblk)]],
                         stage2.at[pl.ds(bsel * kblk, kblk)], semg)

    def gather_wait():
        pltpu.make_async_copy(m2d.at[fsrc.at[pl.ds(0, kblk)]],
                              stage2.at[pl.ds(0, kblk)], semg).wait()

    def consume(sel, nvec):
        cnt = _filter_chunk(sch2, dch2, sel * ECH, fsrc, fdst, nvec,
                            lo, hi, kblk, rows)
        nb = (cnt + kblk - 1) // kblk

        @pl.when(nb > 0)
        def _():
            gather_issue(0, 0)

        def gblk(b, _):
            gather_wait()

            @pl.when(b + 1 < nb)
            def _():
                gather_issue(b + 1, (b + 1) % 2)

            bsel = b % 2
            nrows = jnp.minimum(kblk, cnt - b * kblk)

            @plsc.parallel_loop(0, nrows, unroll=4)
            def _(r):
                base_w = fdst[pl.ds(b * kblk + r, 16)][0] * d
                for k in range(d // 16):
                    v = stage2[bsel * kblk + r, pl.ds(k * 16, 16)]
                    plsc.addupdate(acc1.at[pl.ds(base_w + k * 16, 16)], v)
            return 0
        lax.fori_loop(0, nb, gblk, 0)

    pltpu.async_copy(src_hbm.at[pl.ds(0, ECH)], sch2.at[pl.ds(0, ECH)], sem)
    pltpu.async_copy(dst_hbm.at[pl.ds(0, ECH)], dch2.at[pl.ds(0, ECH)], sem)

    def chunk_loop(ci, _):
        pltpu.make_async_copy(src_hbm.at[pl.ds(0, ECH)],
                              sch2.at[pl.ds(0, ECH)], sem).wait()
        pltpu.make_async_copy(src_hbm.at[pl.ds(0, ECH)],
                              sch2.at[pl.ds(0, ECH)], sem).wait()

        @pl.when(ci + 1 < NCH)
        def _():
            nxt = ((ci + 1) % 2) * ECH
            pltpu.async_copy(src_hbm.at[pl.ds((ci + 1) * ECH, ECH)],
                             sch2.at[pl.ds(nxt, ECH)], sem)
            pltpu.async_copy(dst_hbm.at[pl.ds((ci + 1) * ECH, ECH)],
                             dch2.at[pl.ds(nxt, ECH)], sem)

        consume(ci % 2, ECH // 16)
        return 0
    lax.fori_loop(0, NCH, chunk_loop, 0)

    if ECH_TAIL:
        pltpu.sync_copy(src_hbm.at[pl.ds(NCH * ECH, ECH_TAIL)],
                        sch2.at[pl.ds(0, ECH_TAIL)])
        pltpu.sync_copy(dst_hbm.at[pl.ds(NCH * ECH, ECH_TAIL)],
                        dch2.at[pl.ds(0, ECH_TAIL)])
        consume(0, ECH_TAIL // 16)

    pltpu.sync_copy(acc1.at[pl.ds(0, words)], out1d.at[pl.ds(lo * d, words)])


@functools.partial(
    pl.kernel,
    out_type=[jax.ShapeDtypeStruct((N * DIN,), jnp.float32),
              jax.ShapeDtypeStruct((NC * N,), jnp.float32)],
    mesh=_mesh,
    compiler_params=_scp,
    scratch_types=[
        pltpu.VMEM((2 * ECH,), jnp.int32),
        pltpu.VMEM((2 * ECH,), jnp.int32),
        pltpu.VMEM((FCAP,), jnp.int32),
        pltpu.VMEM((FCAP,), jnp.int32),
        pltpu.VMEM((1024,), jnp.int32),
        pltpu.VMEM((1024,), jnp.int32),
        pltpu.VMEM((1024,), jnp.float32),
        pltpu.VMEM((2 * K0, DIN), jnp.float32),
        pltpu.VMEM(((R0 + 1) * DIN,), jnp.float32),
        pltpu.SemaphoreType.DMA,
        pltpu.SemaphoreType.DMA,
        pltpu.VMEM_SHARED((N,), jnp.float32),
    ],
)
def _agg0_sc(src_hbm, dst_hbm, m2d, m1d, dis_hbm, out1d, gp_hbm,
             sch2, dch2, fsrc, fdst, gsrcb, gdstb, gval, stage2, acc1,
             sem, semg, gs_sh):
    c = lax.axis_index("c")
    s = lax.axis_index("s")
    g = c * NS + s
    iota = _iota16()

    @pl.when(s == 0)
    def _():
        pltpu.sync_copy(dis_hbm, acc1.at[pl.ds(0, N)])
        pltpu.sync_copy(acc1.at[pl.ds(0, N)], gs_sh)

    plsc.subcore_barrier()

    gbase = g * EPW
    for off, nreal in ((0, 1024), (1024, 1024), (2048, 1024), (3072, 1024),
                       (4096, EPW - 4096)):
        npad16 = (nreal + 15) // 16 * 16
        pltpu.sync_copy(src_hbm.at[pl.ds(gbase + off, nreal)],
                        gsrcb.at[pl.ds(0, nreal)])
        pltpu.sync_copy(dst_hbm.at[pl.ds(gbase + off, nreal)],
                        gdstb.at[pl.ds(0, nreal)])
        if nreal % 16:
            tb = nreal // 16 * 16
            tm = (iota + tb) < nreal
            gsrcb[pl.ds(tb, 16)] = jnp.where(tm, gsrcb[pl.ds(tb, 16)], 0)
            gdstb[pl.ds(tb, 16)] = jnp.where(tm, gdstb[pl.ds(tb, 16)], 0)
        pltpu.sync_copy(dis_hbm.at[gdstb.at[pl.ds(0, npad16)]],
                        gval.at[pl.ds(0, npad16)])
        if nreal % 16:
            tb = nreal // 16 * 16
            tm = (iota + tb) < nreal
            gval[pl.ds(tb, 16)] = jnp.where(tm, gval[pl.ds(tb, 16)], 0.0)
        pltpu.sync_copy(gval.at[pl.ds(0, npad16)],
                        gs_sh.at[gsrcb.at[pl.ds(0, npad16)]], add=True)

    lo = g * R0

    @pl.when(g < NW - 1)
    def _():
        _agg_body(src_hbm, dst_hbm, m2d, m1d, out1d, sch2, dch2,
                  fsrc, fdst, stage2, acc1, sem, semg, lo, R0, DIN, K0)

    @pl.when(g == NW - 1)
    def _():
        _agg_body(src_hbm, dst_hbm, m2d, m1d, out1d, sch2, dch2,
                  fsrc, fdst, stage2, acc1, sem, semg, lo, R0_LAST, DIN, K0)

    plsc.subcore_barrier()

    @pl.when(s == 0)
    def _():
        pltpu.sync_copy(gs_sh, acc1.at[pl.ds(0, N)])
        pltpu.sync_copy(acc1.at[pl.ds(0, N)], gp_hbm.at[pl.ds(c * N, N)])


@functools.partial(
    pl.kernel,
    out_type=jax.ShapeDtypeStruct((N * H,), jnp.float32),
    mesh=_mesh,
    compiler_params=_scp,
    scratch_types=[
        pltpu.VMEM((2 * ECH,), jnp.int32),
        pltpu.VMEM((2 * ECH,), jnp.int32),
        pltpu.VMEM((FCAP,), jnp.int32),
        pltpu.VMEM((FCAP,), jnp.int32),
        pltpu.VMEM((2 * K1, H), jnp.float32),
        pltpu.VMEM(((R1 + 1) * H,), jnp.float32),
        pltpu.SemaphoreType.DMA,
        pltpu.SemaphoreType.DMA,
    ],
)
def _agg1_sc(src_hbm, dst_hbm, m2d, m1d, out1d,
             sch2, dch2, fsrc, fdst, stage2, acc1, sem, semg):
    c = lax.axis_index("c")
    s = lax.axis_index("s")
    g = c * NS + s

    _agg_body(src_hbm, dst_hbm, m2d, m1d, out1d, sch2, dch2,
              fsrc, fdst, stage2, acc1, sem, semg, g * R1, R1, H, K1)

    q = g + NW
    lo = q * R1

    @pl.when(g < NW - 1)
    def _():
        _agg_body(src_hbm, dst_hbm, m2d, m1d, out1d, sch2, dch2,
                  fsrc, fdst, stage2, acc1, sem, semg, lo, R1, H, K1)

    @pl.when(g == NW - 1)
    def _():
        _agg_body(src_hbm, dst_hbm, m2d, m1d, out1d, sch2, dch2,
                  fsrc, fdst, stage2, acc1, sem, semg, lo, R1_LAST, H, K1)


_BN = 1000
_GRID = N // _BN


def _dis_m0_body(degp_ref, x_ref, dis_ref, m0_ref):
    dv = lax.rsqrt(degp_ref[0] + degp_ref[1] + 1.0)
    dis_ref[...] = dv
    m0_ref[...] = x_ref[...] * dv


def _dis_m0_tc(degp_r, x):
    return pl.pallas_call(
        _dis_m0_body,
        grid=(_GRID,),
        in_specs=[
            pl.BlockSpec((NC, _BN, 1), lambda i: (0, i, 0)),
            pl.BlockSpec((_BN, DIN), lambda i: (i, 0)),
        ],
        out_specs=[
            pl.BlockSpec((_BN, 1), lambda i: (i, 0)),
            pl.BlockSpec((_BN, DIN), lambda i: (i, 0)),
        ],
        out_shape=[jax.ShapeDtypeStruct((N, 1), jnp.float32),
                   jax.ShapeDtypeStruct((N, DIN), jnp.float32)],
    )(degp_r, x)


def _layer0_body(agg0_ref, dis_ref, w0_ref, b0_ref, m1_ref):
    y = agg0_ref[...] * dis_ref[...]
    h = lax.dot_general(y, w0_ref[...], (((1,), (1,)), ((), ())),
                        preferred_element_type=jnp.float32)
    h = jnp.maximum(h + b0_ref[...], 0.0)
    m1_ref[...] = h * dis_ref[...]


def _layer0_tc(agg0, dis2, W0, b0r):
    return pl.pallas_call(
        _layer0_body,
        grid=(_GRID,),
        in_specs=[
            pl.BlockSpec((_BN, DIN), lambda i: (i, 0)),
            pl.BlockSpec((_BN, 1), lambda i: (i, 0)),
            pl.BlockSpec((H, DIN), lambda i: (0, 0)),
            pl.BlockSpec((1, H), lambda i: (0, 0)),
        ],
        out_specs=pl.BlockSpec((_BN, H), lambda i: (i, 0)),
        out_shape=jax.ShapeDtypeStruct((N, H), jnp.float32),
    )(agg0, dis2, W0, b0r)


def _final_body(agg1_ref, dis_ref, gp_ref, w1_ref, b1_ref, w2_ref, b2_ref,
                wl_ref, bl_ref, out_ref, emb_acc):
    i = pl.program_id(0)
    dv = dis_ref[...]
    y = agg1_ref[...] * dv
    h1 = lax.dot_general(y, w1_ref[...], (((1,), (1,)), ((), ())),
                         preferred_element_type=jnp.float32)
    h1 = jnp.maximum(h1 + b1_ref[...], 0.0)
    cv = dv * (gp_ref[0] + gp_ref[1] - dv)
    part = lax.dot_general(cv, h1, (((0,), (0,)), ((), ())),
                           preferred_element_type=jnp.float32)

    @pl.when(i == 0)
    def _():
        emb_acc[...] = part

    @pl.when(i > 0)
    def _():
        emb_acc[...] = emb_acc[...] + part

    @pl.when(i == _GRID - 1)
    def _():
        emb = emb_acc[...] * (1.0 / N)
        t = lax.dot_general(emb, w2_ref[...], (((1,), (1,)), ((), ())),
                            preferred_element_type=jnp.float32) + b2_ref[...]
        o = lax.dot_general(t, wl_ref[...], (((1,), (1,)), ((), ())),
                            preferred_element_type=jnp.float32) + bl_ref[...]
        out_ref[...] = o


def _final_tc(agg1, dis2, gp_r, W1, b1r, W2, b2r, Wl, blr):
    return pl.pallas_call(
        _final_body,
        grid=(_GRID,),
        in_specs=[
            pl.BlockSpec((_BN, H), lambda i: (i, 0)),
            pl.BlockSpec((_BN, 1), lambda i: (i, 0)),
            pl.BlockSpec((NC, _BN, 1), lambda i: (0, i, 0)),
            pl.BlockSpec((H, H), lambda i: (0, 0)),
            pl.BlockSpec((1, H), lambda i: (0, 0)),
            pl.BlockSpec((H, H), lambda i: (0, 0)),
            pl.BlockSpec((1, H), lambda i: (0, 0)),
            pl.BlockSpec((NCLS, H), lambda i: (0, 0)),
            pl.BlockSpec((1, NCLS), lambda i: (0, 0)),
        ],
        out_specs=pl.BlockSpec((1, NCLS), lambda i: (0, 0)),
        out_shape=jax.ShapeDtypeStruct((1, NCLS), jnp.float32),
        scratch_shapes=[pltpu.VMEM((1, H), jnp.float32)],
    )(agg1, dis2, gp_r, W1, b1r, W2, b2r, Wl, blr)


def kernel(x, edge_index, W0, b0, W1, b1, W2, b2, Wl, bl):
    src = edge_index[0]
    dst = edge_index[1]

    degp = _deg_sc(dst)
    dis2, m0 = _dis_m0_tc(degp.reshape(NC, N, 1), x)
    dis = dis2.reshape(N)
    agg0f, gp = _agg0_sc(src, dst, m0, m0.reshape(N * DIN), dis)
    m1 = _layer0_tc(agg0f.reshape(N, DIN), dis2, W0, b0.reshape(1, H))
    agg1f = _agg1_sc(src, dst, m1, m1.reshape(N * H))
    out = _final_tc(agg1f.reshape(N, H), dis2, gp.reshape(NC, N, 1),
                    W1, b1.reshape(1, H), W2, b2.reshape(1, H),
                    Wl, bl.reshape(1, NCLS))
    return out


# speedup vs baseline: 1.2021x; 1.2021x over previous
import functools

import jax
import jax.numpy as jnp
from jax import lax
from jax.experimental import pallas as pl
from jax.experimental.pallas import tpu as pltpu
from jax.experimental.pallas import tpu_sc as plsc

N = 10000
E = 160000
DIN = 256
H = 512
NCLS = 10

NC = 2
NS = 16
NW = NC * NS
EPW = E // NW
ECH = 4096
NCH_FULL = E // ECH
ECH_TAIL = E - NCH_FULL * ECH
FCAP = ECH + 64

R0 = 313
R0_LAST = N - 31 * R0
R1 = 157
R1_LAST = N - 63 * R1
K0 = 32
K1 = 16
NCH = E // ECH

_mesh = plsc.VectorSubcoreMesh(core_axis_name="c", subcore_axis_name="s")
_scp = pltpu.CompilerParams(needs_layout_passes=False)


def _iota16():
    return lax.broadcasted_iota(jnp.int32, (16,), 0)


@functools.partial(
    pl.kernel,
    out_type=jax.ShapeDtypeStruct((NC * N,), jnp.float32),
    mesh=_mesh,
    compiler_params=_scp,
    scratch_types=[
        pltpu.VMEM((EPW + 24,), jnp.int32),
        pltpu.VMEM((EPW + 24,), jnp.float32),
        pltpu.VMEM_SHARED((N,), jnp.float32),
    ],
)
def _deg_sc(dst_hbm, degp_hbm, idx_v, val_v, acc_sh):
    c = lax.axis_index("c")
    s = lax.axis_index("s")
    wid = c * NS + s
    npad = EPW + 24
    iota = _iota16()

    def zfill(i, _):
        val_v[pl.ds(i * 16, 16)] = jnp.zeros((16,), jnp.float32)
        return 0
    lax.fori_loop(0, npad // 16, zfill, 0)

    @pl.when(s < 2)
    def _():
        pltpu.sync_copy(val_v.at[pl.ds(0, EPW)], acc_sh.at[pl.ds(s * EPW, EPW)])

    plsc.subcore_barrier()

    pltpu.sync_copy(dst_hbm.at[pl.ds(wid * EPW, EPW)], idx_v.at[pl.ds(0, EPW)])

    def vfill(i, _):
        pos = iota + i * 16
        val_v[pl.ds(i * 16, 16)] = jnp.where(pos < EPW, 1.0, 0.0).astype(jnp.float32)
        return 0
    lax.fori_loop(0, npad // 16, vfill, 0)
    for off in range(EPW // 16 * 16, npad, 16):
        pos = iota + off
        old = idx_v[pl.ds(off, 16)]
        idx_v[pl.ds(off, 16)] = jnp.where(pos < EPW, old, 0)

    pltpu.sync_copy(val_v, acc_sh.at[idx_v], add=True)
    plsc.subcore_barrier()

    @pl.when(s < 2)
    def _():
        pltpu.sync_copy(acc_sh.at[pl.ds(s * EPW, EPW)], val_v.at[pl.ds(0, EPW)])
        pltpu.sync_copy(val_v.at[pl.ds(0, EPW)],
                        degp_hbm.at[pl.ds(c * N + s * EPW, EPW)])


def _filter_chunk(schunk, dchunk, base, fsrc, fdst, nvec, lo, hi, kpad,
                  trash):
    iota = _iota16()

    @plsc.parallel_loop(0, nvec, unroll=4, carry=jnp.int32(0))
    def cnt(i, cnt):
        d = dchunk[pl.ds(base + i * 16, 16)]
        sv = schunk[pl.ds(base + i * 16, 16)]
        m = (d >= lo) & (d < hi)
        mi = m.astype(jnp.int32)
        inc = plsc.cumsum(mi)
        pos = cnt + inc - mi
        plsc.store_scatter(fdst, [pos], d - lo, mask=m)
        plsc.store_scatter(fsrc, [pos], sv, mask=m)
        return cnt + plsc.all_reduce_population_count(m)[0]
    for t in range(kpad // 16):
        fsrc[pl.ds(cnt + t * 16, 16)] = iota
        fdst[pl.ds(cnt + t * 16, 16)] = jnp.full((16,), trash, jnp.int32)
    return cnt


def _agg_body(src_hbm, dst_hbm, m2d, m1d, out1d, sch2, dch2, fsrc, fdst,
              stage2, acc1, sem, semg, lo, rows, d, kblk):
    words = rows * d
    pltpu.sync_copy(m1d.at[pl.ds(lo * d, words)], acc1.at[pl.ds(0, words)])
    hi = lo + rows

    def gather_issue(b, bsel):
        pltpu.async_copy(m2d.at[fsrc.at[pl.ds(b * kblk, kblk)]],
                         stage2.at[pl.ds(bsel * kblk, kblk)], semg)

    def gather_wait():
        pltpu.make_async_copy(m2d.at[fsrc.at[pl.ds(0, kblk)]],
                              stage2.at[pl.ds(0, kblk)], semg).wait()

    def consume(sel, nvec):
        cnt = _filter_chunk(sch2, dch2, sel * ECH, fsrc, fdst, nvec,
                            lo, hi, kblk, rows)
        nb = (cnt + kblk - 1) // kblk

        @pl.when(nb > 0)
        def _():
            gather_issue(0, 0)

        def gblk(b, _):
            gather_wait()

            @pl.when(b + 1 < nb)
            def _():
                gather_issue(b + 1, (b + 1) % 2)

            bsel = b % 2
            nrows = jnp.minimum(kblk, cnt - b * kblk)

            @plsc.parallel_loop(0, nrows, unroll=2)
            def _(r):
                base_w = fdst[pl.ds(b * kblk + r, 16)][0] * d
                for k in range(d // 16):
                    v = stage2[bsel * kblk + r, pl.ds(k * 16, 16)]
                    plsc.addupdate(acc1.at[pl.ds(base_w + k * 16, 16)], v)
            return 0
        lax.fori_loop(0, nb, gblk, 0)

    pltpu.async_copy(src_hbm.at[pl.ds(0, ECH)], sch2.at[pl.ds(0, ECH)], sem)
    pltpu.async_copy(dst_hbm.at[pl.ds(0, ECH)], dch2.at[pl.ds(0, ECH)], sem)

    def chunk_loop(ci, _):
        pltpu.make_async_copy(src_hbm.at[pl.ds(0, ECH)],
                              sch2.at[pl.ds(0, ECH)], sem).wait()
        pltpu.make_async_copy(src_hbm.at[pl.ds(0, ECH)],
                              sch2.at[pl.ds(0, ECH)], sem).wait()

        @pl.when(ci + 1 < NCH)
        def _():
            nxt = ((ci + 1) % 2) * ECH
            pltpu.async_copy(src_hbm.at[pl.ds((ci + 1) * ECH, ECH)],
                             sch2.at[pl.ds(nxt, ECH)], sem)
            pltpu.async_copy(dst_hbm.at[pl.ds((ci + 1) * ECH, ECH)],
                             dch2.at[pl.ds(nxt, ECH)], sem)

        consume(ci % 2, ECH // 16)
        return 0
    lax.fori_loop(0, NCH, chunk_loop, 0)

    if ECH_TAIL:
        pltpu.sync_copy(src_hbm.at[pl.ds(NCH * ECH, ECH_TAIL)],
                        sch2.at[pl.ds(0, ECH_TAIL)])
        pltpu.sync_copy(dst_hbm.at[pl.ds(NCH * ECH, ECH_TAIL)],
                        dch2.at[pl.ds(0, ECH_TAIL)])
        consume(0, ECH_TAIL // 16)

    pltpu.sync_copy(acc1.at[pl.ds(0, words)], out1d.at[pl.ds(lo * d, words)])


@functools.partial(
    pl.kernel,
    out_type=[jax.ShapeDtypeStruct((N * DIN,), jnp.float32),
              jax.ShapeDtypeStruct((NC * N,), jnp.float32)],
    mesh=_mesh,
    compiler_params=_scp,
    scratch_types=[
        pltpu.VMEM((2 * ECH,), jnp.int32),
        pltpu.VMEM((2 * ECH,), jnp.int32),
        pltpu.VMEM((FCAP,), jnp.int32),
        pltpu.VMEM((FCAP,), jnp.int32),
        pltpu.VMEM((1024,), jnp.int32),
        pltpu.VMEM((1024,), jnp.int32),
        pltpu.VMEM((1024,), jnp.float32),
        pltpu.VMEM((2 * K0, DIN), jnp.float32),
        pltpu.VMEM(((R0 + 1) * DIN,), jnp.float32),
        pltpu.SemaphoreType.DMA,
        pltpu.SemaphoreType.DMA,
        pltpu.VMEM_SHARED((N,), jnp.float32),
    ],
)
def _agg0_sc(src_hbm, dst_hbm, m2d, m1d, dis_hbm, out1d, gp_hbm,
             sch2, dch2, fsrc, fdst, gsrcb, gdstb, gval, stage2, acc1,
             sem, semg, gs_sh):
    c = lax.axis_index("c")
    s = lax.axis_index("s")
    g = c * NS + s
    iota = _iota16()

    @pl.when(s == 0)
    def _():
        pltpu.sync_copy(dis_hbm, acc1.at[pl.ds(0, N)])
        pltpu.sync_copy(acc1.at[pl.ds(0, N)], gs_sh)

    plsc.subcore_barrier()

    gbase = g * EPW
    for off, nreal in ((0, 1024), (1024, 1024), (2048, 1024), (3072, 1024),
                       (4096, EPW - 4096)):
        npad16 = (nreal + 15) // 16 * 16
        pltpu.sync_copy(src_hbm.at[pl.ds(gbase + off, nreal)],
                        gsrcb.at[pl.ds(0, nreal)])
        pltpu.sync_copy(dst_hbm.at[pl.ds(gbase + off, nreal)],
                        gdstb.at[pl.ds(0, nreal)])
        if nreal % 16:
            tb = nreal // 16 * 16
            tm = (iota + tb) < nreal
            gsrcb[pl.ds(tb, 16)] = jnp.where(tm, gsrcb[pl.ds(tb, 16)], 0)
            gdstb[pl.ds(tb, 16)] = jnp.where(tm, gdstb[pl.ds(tb, 16)], 0)
        pltpu.sync_copy(dis_hbm.at[gdstb.at[pl.ds(0, npad16)]],
                        gval.at[pl.ds(0, npad16)])
        if nreal % 16:
            tb = nreal // 16 * 16
            tm = (iota + tb) < nreal
            gval[pl.ds(tb, 16)] = jnp.where(tm, gval[pl.ds(tb, 16)], 0.0)
        pltpu.sync_copy(gval.at[pl.ds(0, npad16)],
                        gs_sh.at[gsrcb.at[pl.ds(0, npad16)]], add=True)

    lo = g * R0

    @pl.when(g < NW - 1)
    def _():
        _agg_body(src_hbm, dst_hbm, m2d, m1d, out1d, sch2, dch2,
                  fsrc, fdst, stage2, acc1, sem, semg, lo, R0, DIN, K0)

    @pl.when(g == NW - 1)
    def _():
        _agg_body(src_hbm, dst_hbm, m2d, m1d, out1d, sch2, dch2,
                  fsrc, fdst, stage2, acc1, sem, semg, lo, R0_LAST, DIN, K0)

    plsc.subcore_barrier()

    @pl.when(s == 0)
    def _():
        pltpu.sync_copy(gs_sh, acc1.at[pl.ds(0, N)])
        pltpu.sync_copy(acc1.at[pl.ds(0, N)], gp_hbm.at[pl.ds(c * N, N)])


@functools.partial(
    pl.kernel,
    out_type=jax.ShapeDtypeStruct((N * H,), jnp.float32),
    mesh=_mesh,
    compiler_params=_scp,
    scratch_types=[
        pltpu.VMEM((2 * ECH,), jnp.int32),
        pltpu.VMEM((2 * ECH,), jnp.int32),
        pltpu.VMEM((FCAP,), jnp.int32),
        pltpu.VMEM((FCAP,), jnp.int32),
        pltpu.VMEM((2 * K1, H), jnp.float32),
        pltpu.VMEM(((R1 + 1) * H,), jnp.float32),
        pltpu.SemaphoreType.DMA,
        pltpu.SemaphoreType.DMA,
    ],
)
def _agg1_sc(src_hbm, dst_hbm, m2d, m1d, out1d,
             sch2, dch2, fsrc, fdst, stage2, acc1, sem, semg):
    c = lax.axis_index("c")
    s = lax.axis_index("s")
    g = c * NS + s

    _agg_body(src_hbm, dst_hbm, m2d, m1d, out1d, sch2, dch2,
              fsrc, fdst, stage2, acc1, sem, semg, g * R1, R1, H, K1)

    q = g + NW
    lo = q * R1

    @pl.when(g < NW - 1)
    def _():
        _agg_body(src_hbm, dst_hbm, m2d, m1d, out1d, sch2, dch2,
                  fsrc, fdst, stage2, acc1, sem, semg, lo, R1, H, K1)

    @pl.when(g == NW - 1)
    def _():
        _agg_body(src_hbm, dst_hbm, m2d, m1d, out1d, sch2, dch2,
                  fsrc, fdst, stage2, acc1, sem, semg, lo, R1_LAST, H, K1)


_BN = 1000
_GRID = N // _BN


def _dis_m0_body(degp_ref, x_ref, dis_ref, m0_ref):
    dv = lax.rsqrt(degp_ref[0] + degp_ref[1] + 1.0)
    dis_ref[...] = dv
    m0_ref[...] = x_ref[...] * dv


def _dis_m0_tc(degp_r, x):
    return pl.pallas_call(
        _dis_m0_body,
        grid=(_GRID,),
        in_specs=[
            pl.BlockSpec((NC, _BN, 1), lambda i: (0, i, 0)),
            pl.BlockSpec((_BN, DIN), lambda i: (i, 0)),
        ],
        out_specs=[
            pl.BlockSpec((_BN, 1), lambda i: (i, 0)),
            pl.BlockSpec((_BN, DIN), lambda i: (i, 0)),
        ],
        out_shape=[jax.ShapeDtypeStruct((N, 1), jnp.float32),
                   jax.ShapeDtypeStruct((N, DIN), jnp.float32)],
    )(degp_r, x)


def _layer0_body(agg0_ref, dis_ref, w0_ref, b0_ref, m1_ref):
    y = agg0_ref[...] * dis_ref[...]
    h = lax.dot_general(y, w0_ref[...], (((1,), (1,)), ((), ())),
                        preferred_element_type=jnp.float32)
    h = jnp.maximum(h + b0_ref[...], 0.0)
    m1_ref[...] = h * dis_ref[...]


def _layer0_tc(agg0, dis2, W0, b0r):
    return pl.pallas_call(
        _layer0_body,
        grid=(_GRID,),
        in_specs=[
            pl.BlockSpec((_BN, DIN), lambda i: (i, 0)),
            pl.BlockSpec((_BN, 1), lambda i: (i, 0)),
            pl.BlockSpec((H, DIN), lambda i: (0, 0)),
            pl.BlockSpec((1, H), lambda i: (0, 0)),
        ],
        out_specs=pl.BlockSpec((_BN, H), lambda i: (i, 0)),
        out_shape=jax.ShapeDtypeStruct((N, H), jnp.float32),
    )(agg0, dis2, W0, b0r)


def _final_body(agg1_ref, dis_ref, gp_ref, w1_ref, b1_ref, w2_ref, b2_ref,
                wl_ref, bl_ref, out_ref, emb_acc):
    i = pl.program_id(0)
    dv = dis_ref[...]
    y = agg1_ref[...] * dv
    h1 = lax.dot_general(y, w1_ref[...], (((1,), (1,)), ((), ())),
                         preferred_element_type=jnp.float32)
    h1 = jnp.maximum(h1 + b1_ref[...], 0.0)
    cv = dv * (gp_ref[0] + gp_ref[1] - dv)
    part = lax.dot_general(cv, h1, (((0,), (0,)), ((), ())),
                           preferred_element_type=jnp.float32)

    @pl.when(i == 0)
    def _():
        emb_acc[...] = part

    @pl.when(i > 0)
    def _():
        emb_acc[...] = emb_acc[...] + part

    @pl.when(i == _GRID - 1)
    def _():
        emb = emb_acc[...] * (1.0 / N)
        t = lax.dot_general(emb, w2_ref[...], (((1,), (1,)), ((), ())),
                            preferred_element_type=jnp.float32) + b2_ref[...]
        o = lax.dot_general(t, wl_ref[...], (((1,), (1,)), ((), ())),
                            preferred_element_type=jnp.float32) + bl_ref[...]
        out_ref[...] = o


def _final_tc(agg1, dis2, gp_r, W1, b1r, W2, b2r, Wl, blr):
    return pl.pallas_call(
        _final_body,
        grid=(_GRID,),
        in_specs=[
            pl.BlockSpec((_BN, H), lambda i: (i, 0)),
            pl.BlockSpec((_BN, 1), lambda i: (i, 0)),
            pl.BlockSpec((NC, _BN, 1), lambda i: (0, i, 0)),
            pl.BlockSpec((H, H), lambda i: (0, 0)),
            pl.BlockSpec((1, H), lambda i: (0, 0)),
            pl.BlockSpec((H, H), lambda i: (0, 0)),
            pl.BlockSpec((1, H), lambda i: (0, 0)),
            pl.BlockSpec((NCLS, H), lambda i: (0, 0)),
            pl.BlockSpec((1, NCLS), lambda i: (0, 0)),
        ],
        out_specs=pl.BlockSpec((1, NCLS), lambda i: (0, 0)),
        out_shape=jax.ShapeDtypeStruct((1, NCLS), jnp.float32),
        scratch_shapes=[pltpu.VMEM((1, H), jnp.float32)],
    )(agg1, dis2, gp_r, W1, b1r, W2, b2r, Wl, blr)


def kernel(x, edge_index, W0, b0, W1, b1, W2, b2, Wl, bl):
    src = edge_index[0]
    dst = edge_index[1]

    degp = _deg_sc(dst)
    dis2, m0 = _dis_m0_tc(degp.reshape(NC, N, 1), x)
    dis = dis2.reshape(N)
    agg0f, gp = _agg0_sc(src, dst, m0, m0.reshape(N * DIN), dis)
    m1 = _layer0_tc(agg0f.reshape(N, DIN), dis2, W0, b0.reshape(1, H))
    agg1f = _agg1_sc(src, dst, m1, m1.reshape(N * H))
    out = _final_tc(agg1f.reshape(N, H), dis2, gp.reshape(NC, N, 1),
                    W1, b1.reshape(1, H), W2, b2.reshape(1, H),
                    Wl, bl.reshape(1, NCLS))
    return out
